# baseline (speedup 1.0000x reference)
import functools

import jax
import jax.numpy as jnp
from jax import lax
from jax.experimental import pallas as pl
from jax.experimental.pallas import tpu as pltpu
from jax.experimental.pallas import tpu_sc as plsc

B = 16
TOTAL = 32768
CLS_ID = 101
SEP_ID = 102
PADLEN = 4098
MAXTOK = PADLEN - 2

CW = 128
FRONT = 8
SW = CW + 16
WBUF = 176

_mesh = plsc.VectorSubcoreMesh(core_axis_name="c", subcore_axis_name="s")


@functools.partial(
    pl.kernel,
    out_type=jax.ShapeDtypeStruct((B, PADLEN), jnp.int32),
    mesh=_mesh,
    scratch_types=[
        pltpu.VMEM((32,), jnp.int32),
        pltpu.VMEM((B * WBUF,), jnp.int32),
        pltpu.VMEM((B, CW), jnp.int32),
        pltpu.SemaphoreType.DMA,
    ],
)
def _pad_tiles(cu_hbm, flat_hbm, zeros_hbm, out_hbm, cu_v, sbuf, tv, sem):
    del zeros_hbm
    c = lax.axis_index("c")
    s = lax.axis_index("s")
    w = s * 2 + c
    cg0 = pl.multiple_of(w * CW, CW)

    pltpu.sync_copy(cu_hbm, cu_v.at[pl.ds(0, B + 1)])
    lane = lax.iota(jnp.int32, 16)

    def window(i):
        start_i = cu_v[pl.ds(i, 16)][0]
        src_lo = start_i + cg0 - 1
        abase = jnp.clip((jnp.maximum(src_lo, 0) // 8) * 8, 0, TOTAL - SW)
        return src_lo, pl.multiple_of(abase, 8)

    def fire(i, carry):
        _, abase = window(i)
        dst = pl.multiple_of(i * WBUF + FRONT, 8)
        pltpu.async_copy(flat_hbm.at[pl.ds(abase, SW)],
                         sbuf.at[pl.ds(dst, SW)], sem)
        return carry

    lax.fori_loop(0, B, fire, 0)

    pltpu.make_async_copy(flat_hbm.at[pl.ds(0, B * SW)],
                          sbuf.at[pl.ds(0, B * SW)], sem).wait()

    def rows(lo, hi):
        @plsc.parallel_loop(lo, hi)
        def _(i):
            row(i, 0)

    def row(i, carry):
        start_i = cu_v[pl.ds(i, 16)][0]
        nxt_i = cu_v[pl.ds(i + 1, 16)][0]
        clen_i = jnp.minimum(nxt_i - start_i, MAXTOK)
        src_lo, abase = window(i)
        shift = src_lo - abase + FRONT

        def vec(j, carry2):
            col = cg0 + j * 16 + lane
            base = i * WBUF + jnp.clip(shift + j * 16, 0, WBUF - 16)
            tok = sbuf[pl.ds(base, 16)]
            tv[i, pl.ds(j * 16, 16)] = jnp.where(col <= clen_i, tok, jnp.int32(0))
            return carry2

        lax.fori_loop(0, CW // 16, vec, 0)

        sep_l = clen_i + 1 - cg0
        @pl.when((sep_l >= 0) & (sep_l < CW))
        def _():
            jb = (sep_l // 16) * 16
            v = tv[i, pl.ds(jb, 16)]
            tv[i, pl.ds(jb, 16)] = jnp.where(lane == sep_l - jb,
                                             jnp.int32(SEP_ID), v)

        @pl.when(w == 0)
        def _():
            v = tv[i, pl.ds(0, 16)]
            tv[i, pl.ds(0, 16)] = jnp.where(lane == 0, jnp.int32(CLS_ID), v)

        return carry

    rows(0, B)
    pltpu.sync_copy(tv.at[pl.ds(0, 8)], out_hbm.at[pl.ds(0, 8), pl.ds(cg0, CW)])
    pltpu.sync_copy(tv.at[pl.ds(8, 8)], out_hbm.at[pl.ds(8, 8), pl.ds(cg0, CW)])


def kernel(flat_ids, cu_seqlens):
    type_ids = jnp.zeros((B, PADLEN), jnp.int32)
    main = _pad_tiles(cu_seqlens, flat_ids, type_ids)
    starts = cu_seqlens[:B]
    clens = jnp.minimum(cu_seqlens[1:] - cu_seqlens[:-1], MAXTOK)
    tok = flat_ids[jnp.clip(starts + MAXTOK - 1, 0, TOTAL - 1)]
    c0 = jnp.where(clens == MAXTOK, tok,
                   jnp.where(clens == MAXTOK - 1, SEP_ID, 0)).astype(jnp.int32)
    c1 = jnp.where(clens == MAXTOK, SEP_ID, 0).astype(jnp.int32)
    tail = jnp.stack([c0, c1], axis=1)
    padded = lax.dynamic_update_slice(main, tail, (0, MAXTOK))
    return padded, type_ids


# speedup vs baseline: 1.0023x; 1.0023x over previous
import functools

import jax
import jax.numpy as jnp
from jax import lax
from jax.experimental import pallas as pl
from jax.experimental.pallas import tpu as pltpu
from jax.experimental.pallas import tpu_sc as plsc

B = 16
TOTAL = 32768
CLS_ID = 101
SEP_ID = 102
PADLEN = 4098
MAXTOK = PADLEN - 2

CW = 128
FRONT = 8
SW = CW + 16
WBUF = 176

_mesh = plsc.VectorSubcoreMesh(core_axis_name="c", subcore_axis_name="s")


@functools.partial(
    pl.kernel,
    out_type=jax.ShapeDtypeStruct((B, PADLEN), jnp.int32),
    mesh=_mesh,
    scratch_types=[
        pltpu.VMEM((32,), jnp.int32),
        pltpu.VMEM((B * WBUF,), jnp.int32),
        pltpu.VMEM((B, CW), jnp.int32),
        pltpu.SemaphoreType.DMA,
    ],
)
def _pad_tiles(cu_hbm, flat_hbm, zeros_hbm, out_hbm, cu_v, sbuf, tv, sem):
    del zeros_hbm
    c = lax.axis_index("c")
    s = lax.axis_index("s")
    w = s * 2 + c
    cg0 = pl.multiple_of(w * CW, CW)

    pltpu.sync_copy(cu_hbm, cu_v.at[pl.ds(0, B + 1)])
    lane = lax.iota(jnp.int32, 16)

    def window(i):
        start_i = cu_v[pl.ds(i, 16)][0]
        src_lo = start_i + cg0 - 1
        abase = jnp.clip((jnp.maximum(src_lo, 0) // 8) * 8, 0, TOTAL - SW)
        return src_lo, pl.multiple_of(abase, 8)

    def fire(i, carry):
        _, abase = window(i)
        dst = pl.multiple_of(i * WBUF + FRONT, 8)
        pltpu.async_copy(flat_hbm.at[pl.ds(abase, SW)],
                         sbuf.at[pl.ds(dst, SW)], sem)
        return carry

    lax.fori_loop(0, B, fire, 0)

    pltpu.make_async_copy(flat_hbm.at[pl.ds(0, B * SW)],
                          sbuf.at[pl.ds(0, B * SW)], sem).wait()

    def rows(lo, hi):
        lax.fori_loop(lo, hi, row, 0)

    def row(i, carry):
        start_i = cu_v[pl.ds(i, 16)][0]
        nxt_i = cu_v[pl.ds(i + 1, 16)][0]
        clen_i = jnp.minimum(nxt_i - start_i, MAXTOK)
        src_lo, abase = window(i)
        shift = src_lo - abase + FRONT

        def vec(j, carry2):
            col = cg0 + j * 16 + lane
            base = i * WBUF + jnp.clip(shift + j * 16, 0, WBUF - 16)
            tok = sbuf[pl.ds(base, 16)]
            tv[i, pl.ds(j * 16, 16)] = jnp.where(col <= clen_i, tok, jnp.int32(0))
            return carry2

        lax.fori_loop(0, CW // 16, vec, 0)

        sep_l = clen_i + 1 - cg0
        @pl.when((sep_l >= 0) & (sep_l < CW))
        def _():
            jb = (sep_l // 16) * 16
            v = tv[i, pl.ds(jb, 16)]
            tv[i, pl.ds(jb, 16)] = jnp.where(lane == sep_l - jb,
                                             jnp.int32(SEP_ID), v)

        @pl.when(w == 0)
        def _():
            v = tv[i, pl.ds(0, 16)]
            tv[i, pl.ds(0, 16)] = jnp.where(lane == 0, jnp.int32(CLS_ID), v)

        return carry

    rows(0, B)
    pltpu.sync_copy(tv.at[pl.ds(0, 8)], out_hbm.at[pl.ds(0, 8), pl.ds(cg0, CW)])
    pltpu.sync_copy(tv.at[pl.ds(8, 8)], out_hbm.at[pl.ds(8, 8), pl.ds(cg0, CW)])


def kernel(flat_ids, cu_seqlens):
    type_ids = jnp.zeros((B, PADLEN), jnp.int32)
    main = _pad_tiles(cu_seqlens, flat_ids, type_ids)
    starts = cu_seqlens[:B]
    clens = jnp.minimum(cu_seqlens[1:] - cu_seqlens[:-1], MAXTOK)
    tok = flat_ids[jnp.clip(starts + MAXTOK - 1, 0, TOTAL - 1)]
    c0 = jnp.where(clens == MAXTOK, tok,
                   jnp.where(clens == MAXTOK - 1, SEP_ID, 0)).astype(jnp.int32)
    c1 = jnp.where(clens == MAXTOK, SEP_ID, 0).astype(jnp.int32)
    tail = jnp.stack([c0, c1], axis=1)
    padded = lax.dynamic_update_slice(main, tail, (0, MAXTOK))
    return padded, type_ids
